# Initial kernel scaffold; baseline (speedup 1.0000x reference)
import jax
import jax.numpy as jnp
from jax.experimental import pallas as pl


def kernel(x_cat, x_cont, emb_tables, W0, b0, W1, b1, bn_w, bn_b):
    raise NotImplementedError("write your pallas kernel here")


# speedup vs baseline: 8.0150x; 8.0150x over previous
import functools

import jax
import jax.numpy as jnp
from jax import lax
from jax.experimental import pallas as pl
from jax.experimental.pallas import tpu as pltpu
from jax.experimental.pallas import tpu_sc as plsc

B, F, V, D, NCF = 16384, 26, 100000, 32, 13
H0, H1 = 512, 256

NW = 32
ROWS_PER_W = B * F // NW
CHUNK = 128
GROUP = 8
N_CHUNK = ROWS_PER_W // CHUNK
N_GROUP = N_CHUNK // GROUP
TB = 1024


def _gather_body(table_hbm, idx_hbm, out_hbm, idx_v, buf, sem):
    nc = 2
    wid = lax.axis_index("s") * nc + lax.axis_index("c")
    pltpu.sync_copy(idx_hbm.at[wid], idx_v)
    out_base = wid * ROWS_PER_W

    def group(g, carry):
        handles = []
        for j in range(GROUP):
            c = g * GROUP + j
            handles.append(pltpu.async_copy(
                table_hbm.at[idx_v.at[c]],
                buf.at[pl.ds(j * CHUNK, CHUNK)],
                sem))
        for h in handles:
            h.wait()
        pltpu.sync_copy(
            buf, out_hbm.at[pl.ds(out_base + g * (GROUP * CHUNK), GROUP * CHUNK)])
        return carry

    lax.fori_loop(0, N_GROUP, group, 0)


def _sc_gather(table_flat, idx3):
    mesh = plsc.VectorSubcoreMesh(core_axis_name="c", subcore_axis_name="s")
    fn = functools.partial(
        pl.kernel,
        mesh=mesh,
        out_type=jax.ShapeDtypeStruct((B * F, D), jnp.float32),
        scratch_types=[
            pltpu.VMEM((N_CHUNK, CHUNK), jnp.int32),
            pltpu.VMEM((GROUP * CHUNK, D), jnp.float32),
            pltpu.SemaphoreType.DMA,
        ],
        compiler_params=pltpu.CompilerParams(use_tc_tiling_on_sc=False),
    )(_gather_body)
    return fn(table_flat, idx3)


def _mlp_body(xc_ref, bnw_ref, bnb_ref, emb_ref, w0e_ref, w0c_ref, b0_ref,
              w1_ref, b1_ref, out_ref, so_ref):
    i = pl.program_id(0)

    @pl.when(i == 0)
    def _():
        xc = xc_ref[...]
        mean = jnp.mean(xc, axis=0)
        var = jnp.mean(xc * xc, axis=0) - mean * mean
        s = bnw_ref[0, :] * lax.rsqrt(var + 1e-5)
        so_ref[0, :] = s
        so_ref[1, :] = bnb_ref[0, :] - mean * s

    s = so_ref[0, :]
    o = so_ref[1, :]
    xcn = xc_ref[pl.ds(i * TB, TB), :] * s[None, :] + o[None, :]
    xe = emb_ref[...].astype(jnp.bfloat16)
    h = lax.dot_general(xe, w0e_ref[...], (((1,), (1,)), ((), ())),
                        preferred_element_type=jnp.float32)
    h = h + lax.dot_general(xcn, w0c_ref[...], (((1,), (1,)), ((), ())),
                            preferred_element_type=jnp.float32)
    h = jnp.maximum(h + b0_ref[0, :][None, :], 0.0).astype(jnp.bfloat16)
    y = lax.dot_general(h, w1_ref[...], (((1,), (1,)), ((), ())),
                        preferred_element_type=jnp.float32)
    out_ref[...] = jnp.maximum(y + b1_ref[0, :][None, :], 0.0)


def _mlp(xc, bnw, bnb, emb2, w0e, w0c, b0r, w1b, b1r):
    return pl.pallas_call(
        _mlp_body,
        grid=(B // TB,),
        in_specs=[
            pl.BlockSpec((B, NCF), lambda i: (0, 0)),
            pl.BlockSpec((1, NCF), lambda i: (0, 0)),
            pl.BlockSpec((1, NCF), lambda i: (0, 0)),
            pl.BlockSpec((TB, F * D), lambda i: (i, 0)),
            pl.BlockSpec((H0, F * D), lambda i: (0, 0)),
            pl.BlockSpec((H0, NCF), lambda i: (0, 0)),
            pl.BlockSpec((1, H0), lambda i: (0, 0)),
            pl.BlockSpec((H1, H0), lambda i: (0, 0)),
            pl.BlockSpec((1, H1), lambda i: (0, 0)),
        ],
        out_specs=pl.BlockSpec((TB, H1), lambda i: (i, 0)),
        out_shape=jax.ShapeDtypeStruct((B, H1), jnp.float32),
        scratch_shapes=[pltpu.VMEM((2, NCF), jnp.float32)],
    )(xc, bnw, bnb, emb2, w0e, w0c, b0r, w1b, b1r)


def kernel(x_cat, x_cont, emb_tables, W0, b0, W1, b1, bn_w, bn_b):
    table_flat = emb_tables.reshape(F * V, D)
    gidx = x_cat + (jnp.arange(F, dtype=x_cat.dtype) * V)[None, :]
    idx3 = gidx.reshape(NW, N_CHUNK, CHUNK)
    emb = _sc_gather(table_flat, idx3)
    emb2 = emb.reshape(B, F * D)
    w0e = W0[:, :F * D].astype(jnp.bfloat16)
    w0c = W0[:, F * D:]
    w1b = W1.astype(jnp.bfloat16)
    return _mlp(x_cont, bn_w.reshape(1, -1), bn_b.reshape(1, -1), emb2,
                w0e, w0c, b0.reshape(1, -1), w1b, b1.reshape(1, -1))
